# baseline (speedup 1.0000x reference)
import functools

import jax
import jax.numpy as jnp
from jax import lax
from jax.experimental import pallas as pl
from jax.experimental.pallas import tpu as pltpu
from jax.experimental.pallas import tpu_sc as plsc

NC = 2
NS = 16
NW = NC * NS


def _gather_body(uidx_hbm, iidx_hbm, t_umlp, t_imlp, t_umf, t_imf,
                 o_umlp, o_imlp, o_umf, o_imf,
                 uidx_v, iidx_v, r_umlp, r_imlp, r_umf, r_imf, sem,
                 *, b_per_w):
    wid = lax.axis_index("s") * NC + lax.axis_index("c")
    base = wid * b_per_w
    pltpu.sync_copy(uidx_hbm.at[pl.ds(base, b_per_w)], uidx_v)
    pltpu.sync_copy(iidx_hbm.at[pl.ds(base, b_per_w)], iidx_v)
    c0 = pltpu.async_copy(t_umlp.at[uidx_v], r_umlp, sem)
    c1 = pltpu.async_copy(t_imlp.at[iidx_v], r_imlp, sem)
    c2 = pltpu.async_copy(t_umf.at[uidx_v], r_umf, sem)
    c3 = pltpu.async_copy(t_imf.at[iidx_v], r_imf, sem)
    c0.wait()
    c1.wait()
    c2.wait()
    c3.wait()
    pltpu.sync_copy(r_umlp, o_umlp.at[pl.ds(base, b_per_w)])
    pltpu.sync_copy(r_imlp, o_imlp.at[pl.ds(base, b_per_w)])
    pltpu.sync_copy(r_umf, o_umf.at[pl.ds(base, b_per_w)])
    pltpu.sync_copy(r_imf, o_imf.at[pl.ds(base, b_per_w)])


@functools.lru_cache(maxsize=None)
def _make_gather(B, D):
    assert B % (8 * NW) == 0
    b_per_w = B // NW
    mesh = plsc.VectorSubcoreMesh(core_axis_name="c", subcore_axis_name="s",
                                  num_cores=NC, num_subcores=NS)
    f32 = jnp.float32
    out = jax.ShapeDtypeStruct((B, D), f32)
    return pl.kernel(
        functools.partial(_gather_body, b_per_w=b_per_w),
        out_type=(out, out, out, out),
        mesh=mesh,
        scratch_types=[
            pltpu.VMEM((b_per_w,), jnp.int32),
            pltpu.VMEM((b_per_w,), jnp.int32),
            pltpu.VMEM((b_per_w, D), f32),
            pltpu.VMEM((b_per_w, D), f32),
            pltpu.VMEM((b_per_w, D), f32),
            pltpu.VMEM((b_per_w, D), f32),
            pltpu.SemaphoreType.DMA,
        ],
        compiler_params=pltpu.CompilerParams(use_tc_tiling_on_sc=False),
    )


def _mlp_body(ue_ref, ie_ref, um_ref, im_ref, W1_ref, b1_ref, W2_ref, b2_ref,
              W3_ref, b3_ref, Wa_ref, ba_ref, out_ref):
    x = jnp.concatenate([ue_ref[...], ie_ref[...]], axis=1)
    h = jnp.maximum(jnp.dot(x, W1_ref[...],
                            preferred_element_type=jnp.float32) + b1_ref[...], 0.0)
    h = jnp.maximum(jnp.dot(h, W2_ref[...],
                            preferred_element_type=jnp.float32) + b2_ref[...], 0.0)
    h = jnp.maximum(jnp.dot(h, W3_ref[...],
                            preferred_element_type=jnp.float32) + b3_ref[...], 0.0)
    mf = um_ref[...] * im_ref[...]
    v = jnp.concatenate([h, mf], axis=1)
    out_ref[...] = jnp.dot(v, Wa_ref[...],
                           preferred_element_type=jnp.float32) + ba_ref[...]


def kernel(user_indices, item_indices, emb_user_mlp, emb_item_mlp,
           emb_user_mf, emb_item_mf, W1, b1, W2, b2, W3, b3, Wa, ba):
    B = user_indices.shape[0]
    D = emb_user_mlp.shape[1]
    gather = _make_gather(B, D)
    ue_mlp, ie_mlp, ue_mf, ie_mf = gather(
        user_indices.astype(jnp.int32), item_indices.astype(jnp.int32),
        emb_user_mlp, emb_item_mlp, emb_user_mf, emb_item_mf)

    logits = pl.pallas_call(
        _mlp_body,
        out_shape=jax.ShapeDtypeStruct((B, 1), jnp.float32),
    )(ue_mlp, ie_mlp, ue_mf, ie_mf,
      W1, b1.reshape(1, -1), W2, b2.reshape(1, -1), W3, b3.reshape(1, -1),
      Wa, ba.reshape(1, -1))
    return logits
